# baseline (speedup 1.0000x reference)
import functools

import jax
import jax.numpy as jnp
from jax import lax
from jax.experimental import pallas as pl
from jax.experimental.pallas import tpu as pltpu
from jax.experimental.pallas import tpu_sc as plsc

B = 16384
DIM = 128
NC = 2
NS = 16
NW = NC * NS
SPW = B // NW
C = 64
NCHUNK = SPW // C
GROUPS = C // 16

_COS = (
    0.9989871519760842, -0.49624862730581776, 0.0395223027568335,
    -0.0009928615940640857,
)
_SIN = (
    0.999882465186241, -0.1662326327675864, 0.00808644586820865,
    -0.00015325191256653362,
)


def _sincos(ph):
    y = ph * ph
    c = jnp.float32(_COS[-1])
    for k in reversed(_COS[:-1]):
        c = c * y + jnp.float32(k)
    s = jnp.float32(_SIN[-1])
    for k in reversed(_SIN[:-1]):
        s = s * y + jnp.float32(k)
    return s * ph, c


def _body(pos_hbm, er_hbm, ei_hbm, rel_hbm, gam_hbm, out_hbm,
          chunk_pos, hidx, ridx, tidx,
          hre0, hre1, him0, him1, rph0, rph1, tre0, tre1, tim0, tim1,
          gam, outv, stag, sem0, sem1, osem):
    cid = lax.axis_index("c")
    sid = lax.axis_index("s")
    wid = sid * NC + cid
    base = wid * SPW

    row_refs = ((hre0, him0, rph0, tre0, tim0),
                (hre1, him1, rph1, tre1, tim1))
    sems = (sem0, sem1)
    iota = lax.iota(jnp.int32, 16)

    pltpu.sync_copy(gam_hbm, gam)
    neg_gamma = -gam[...]

    def unpack(k):
        pltpu.sync_copy(pos_hbm.at[pl.ds(base + k * C, C)], chunk_pos)
        for g in range(GROUPS):
            rows = iota + jnp.int32(g * 16)
            sl = pl.ds(k * C + g * 16, 16)
            hidx[sl] = plsc.load_gather(
                chunk_pos, [rows, jnp.zeros((16,), jnp.int32)])
            ridx[sl] = plsc.load_gather(
                chunk_pos, [rows, jnp.ones((16,), jnp.int32)])
            tidx[sl] = plsc.load_gather(
                chunk_pos, [rows, jnp.full((16,), 2, jnp.int32)])

    def fetch(k, slot):
        sl = pl.ds(k * C, C)
        hre, him, rph, tre, tim = row_refs[slot]
        sem = sems[slot]
        return [
            pltpu.async_copy(er_hbm.at[hidx.at[sl]], hre, sem),
            pltpu.async_copy(ei_hbm.at[hidx.at[sl]], him, sem),
            pltpu.async_copy(rel_hbm.at[ridx.at[sl]], rph, sem),
            pltpu.async_copy(er_hbm.at[tidx.at[sl]], tre, sem),
            pltpu.async_copy(ei_hbm.at[tidx.at[sl]], tim, sem),
        ]

    def compute(k, slot):
        hre, him, rph, tre, tim = row_refs[slot]

        def sbody(i, _):
            acc = jnp.zeros((16,), jnp.float32)
            for dg in range(DIM // 16):
                sl = pl.ds(dg * 16, 16)
                ph = rph[i, sl]
                a = hre[i, sl]
                b = him[i, sl]
                cre = tre[i, sl]
                cim = tim[i, sl]
                sv, cv = _sincos(ph)
                re = a * cv - b * sv - cre
                im = a * sv + b * cv - cim
                acc = acc + (jnp.abs(re) + jnp.abs(im))
            stag[i, :] = acc
            return jnp.int32(0)

        lax.fori_loop(0, C, sbody, jnp.int32(0), unroll=2)

        for g in range(GROUPS):
            rows = iota + jnp.int32(g * 16)
            tot = neg_gamma
            for r in range(16):
                cols = (iota + jnp.int32(r)) & jnp.int32(15)
                v = plsc.load_gather(stag, [rows, cols])
                tot = tot + v
            outv[slot, pl.ds(g * 16, 16)] = tot
        return pltpu.async_copy(
            outv.at[slot], out_hbm.at[pl.ds(base + k * C, C)], osem)

    unpack(0)
    cps = fetch(0, 0)
    ocps = [None, None]
    for k in range(NCHUNK):
        nxt = None
        if k + 1 < NCHUNK:
            unpack(k + 1)
            nxt = fetch(k + 1, (k + 1) % 2)
        for cp in cps:
            cp.wait()
        if ocps[k % 2] is not None:
            ocps[k % 2].wait()
        ocps[k % 2] = compute(k, k % 2)
        cps = nxt
    for ocp in ocps:
        if ocp is not None:
            ocp.wait()


@jax.jit
def _rotate_scores(pos, er, ei, rel, gam_vec):
    mesh = plsc.VectorSubcoreMesh(
        core_axis_name="c", subcore_axis_name="s", num_cores=NC,
        num_subcores=NS)
    grid = functools.partial(
        pl.kernel,
        out_type=jax.ShapeDtypeStruct((B,), jnp.float32),
        mesh=mesh,
        compiler_params=pltpu.CompilerParams(needs_layout_passes=False),
        scratch_types=(
            [pltpu.VMEM((C, 3), jnp.int32)]
            + [pltpu.VMEM((SPW,), jnp.int32)] * 3
            + [pltpu.VMEM((C, DIM), jnp.float32)] * 10
            + [pltpu.VMEM((16,), jnp.float32),
               pltpu.VMEM((2, C), jnp.float32),
               pltpu.VMEM((C, 16), jnp.float32),
               pltpu.SemaphoreType.DMA,
               pltpu.SemaphoreType.DMA,
               pltpu.SemaphoreType.DMA]
        ),
    )
    return grid(_body)(pos, er, ei, rel, gam_vec)


def kernel(pos_sample, ent_embd, ent_embd_im, rel_embd, gamma):
    gam_vec = jnp.full((16,), gamma, jnp.float32)
    scores = _rotate_scores(
        pos_sample.astype(jnp.int32), ent_embd, ent_embd_im, rel_embd,
        gam_vec)
    return scores.reshape(B, 1)


# speedup vs baseline: 1.0669x; 1.0669x over previous
import functools

import jax
import jax.numpy as jnp
from jax import lax
from jax.experimental import pallas as pl
from jax.experimental.pallas import tpu as pltpu
from jax.experimental.pallas import tpu_sc as plsc

B = 16384
DIM = 128
NC = 2
NS = 16
NW = NC * NS
SPW = B // NW
C = 64
NCHUNK = SPW // C
GROUPS = C // 16

_COS = (
    0.9989871519760842, -0.49624862730581776, 0.0395223027568335,
    -0.0009928615940640857,
)
_SIN = (
    0.999882465186241, -0.1662326327675864, 0.00808644586820865,
    -0.00015325191256653362,
)


def _sincos(ph):
    y = ph * ph
    c = jnp.float32(_COS[-1])
    for k in reversed(_COS[:-1]):
        c = c * y + jnp.float32(k)
    s = jnp.float32(_SIN[-1])
    for k in reversed(_SIN[:-1]):
        s = s * y + jnp.float32(k)
    return s * ph, c


def _body(pos_hbm, er_hbm, ei_hbm, rel_hbm, gam_hbm, out_hbm,
          chunk_pos, hidx, ridx, tidx,
          hre0, hre1, him0, him1, rph0, rph1, tre0, tre1, tim0, tim1,
          gam, outv, stag, sem0, sem1, osem0, osem1, psem0, psem1):
    cid = lax.axis_index("c")
    sid = lax.axis_index("s")
    wid = sid * NC + cid
    base = wid * SPW

    row_refs = ((hre0, him0, rph0, tre0, tim0),
                (hre1, him1, rph1, tre1, tim1))
    sems = (sem0, sem1)
    iota = lax.iota(jnp.int32, 16)

    pltpu.sync_copy(gam_hbm, gam)
    neg_gamma = -gam[...]

    def pos_fetch(k, slot):
        return pltpu.async_copy(
            pos_hbm.at[pl.ds(base + k * C, C)],
            chunk_pos.at[pl.ds(slot * C, C)],
            psem0 if slot == 0 else psem1)

    def unpack(k, slot):
        for g in range(GROUPS):
            rows = iota + jnp.int32(slot * C + g * 16)
            sl = pl.ds(k * C + g * 16, 16)
            hidx[sl] = plsc.load_gather(
                chunk_pos, [rows, jnp.zeros((16,), jnp.int32)])
            ridx[sl] = plsc.load_gather(
                chunk_pos, [rows, jnp.ones((16,), jnp.int32)])
            tidx[sl] = plsc.load_gather(
                chunk_pos, [rows, jnp.full((16,), 2, jnp.int32)])

    def fetch(k, slot):
        sl = pl.ds(k * C, C)
        hre, him, rph, tre, tim = row_refs[slot]
        sem = sems[slot]
        return [
            pltpu.async_copy(er_hbm.at[hidx.at[sl]], hre, sem),
            pltpu.async_copy(ei_hbm.at[hidx.at[sl]], him, sem),
            pltpu.async_copy(rel_hbm.at[ridx.at[sl]], rph, sem),
            pltpu.async_copy(er_hbm.at[tidx.at[sl]], tre, sem),
            pltpu.async_copy(ei_hbm.at[tidx.at[sl]], tim, sem),
        ]

    def compute(k, slot):
        hre, him, rph, tre, tim = row_refs[slot]

        def sbody(i, _):
            acc = jnp.zeros((16,), jnp.float32)
            for dg in range(DIM // 16):
                sl = pl.ds(dg * 16, 16)
                ph = rph[i, sl]
                a = hre[i, sl]
                b = him[i, sl]
                cre = tre[i, sl]
                cim = tim[i, sl]
                sv, cv = _sincos(ph)
                re = a * cv - b * sv - cre
                im = a * sv + b * cv - cim
                acc = acc + (jnp.abs(re) + jnp.abs(im))
            stag[i, :] = acc
            return jnp.int32(0)

        lax.fori_loop(0, C, sbody, jnp.int32(0), unroll=2)

        for g in range(GROUPS):
            rows = iota + jnp.int32(g * 16)
            tot = neg_gamma
            for r in range(16):
                cols = (iota + jnp.int32(r)) & jnp.int32(15)
                v = plsc.load_gather(stag, [rows, cols])
                tot = tot + v
            outv[slot, pl.ds(g * 16, 16)] = tot
        return pltpu.async_copy(
            outv.at[slot], out_hbm.at[pl.ds(base + k * C, C)],
            osem0 if slot == 0 else osem1)

    pcps = [pos_fetch(0, 0), pos_fetch(1, 1)]
    pcps[0].wait()
    unpack(0, 0)
    cps = fetch(0, 0)
    ocps = [None, None]
    for k in range(NCHUNK):
        nxt = None
        if k + 1 < NCHUNK:
            pcps[(k + 1) % 2].wait()
            unpack(k + 1, (k + 1) % 2)
            nxt = fetch(k + 1, (k + 1) % 2)
            if k + 2 < NCHUNK:
                pcps[k % 2] = pos_fetch(k + 2, k % 2)
        for cp in cps:
            cp.wait()
        if ocps[k % 2] is not None:
            ocps[k % 2].wait()
        ocps[k % 2] = compute(k, k % 2)
        cps = nxt
    for ocp in ocps:
        if ocp is not None:
            ocp.wait()


@jax.jit
def _rotate_scores(pos, er, ei, rel, gam_vec):
    mesh = plsc.VectorSubcoreMesh(
        core_axis_name="c", subcore_axis_name="s", num_cores=NC,
        num_subcores=NS)
    grid = functools.partial(
        pl.kernel,
        out_type=jax.ShapeDtypeStruct((B,), jnp.float32),
        mesh=mesh,
        compiler_params=pltpu.CompilerParams(needs_layout_passes=False),
        scratch_types=(
            [pltpu.VMEM((2 * C, 3), jnp.int32)]
            + [pltpu.VMEM((SPW,), jnp.int32)] * 3
            + [pltpu.VMEM((C, DIM), jnp.float32)] * 10
            + [pltpu.VMEM((16,), jnp.float32),
               pltpu.VMEM((2, C), jnp.float32),
               pltpu.VMEM((C, 16), jnp.float32),
               pltpu.SemaphoreType.DMA,
               pltpu.SemaphoreType.DMA,
               pltpu.SemaphoreType.DMA,
               pltpu.SemaphoreType.DMA,
               pltpu.SemaphoreType.DMA,
               pltpu.SemaphoreType.DMA]
        ),
    )
    return grid(_body)(pos, er, ei, rel, gam_vec)


def kernel(pos_sample, ent_embd, ent_embd_im, rel_embd, gamma):
    gam_vec = jnp.full((16,), gamma, jnp.float32)
    scores = _rotate_scores(
        pos_sample.astype(jnp.int32), ent_embd, ent_embd_im, rel_embd,
        gam_vec)
    return scores.reshape(B, 1)


# speedup vs baseline: 1.1031x; 1.0339x over previous
import functools

import jax
import jax.numpy as jnp
from jax import lax
from jax.experimental import pallas as pl
from jax.experimental.pallas import tpu as pltpu
from jax.experimental.pallas import tpu_sc as plsc

B = 16384
DIM = 128
NC = 2
NS = 16
NW = NC * NS
SPW = B // NW
C = 64
NCHUNK = SPW // C
GROUPS = C // 16

_COS = (
    0.9989871519760842, -0.49624862730581776, 0.0395223027568335,
    -0.0009928615940640857,
)
_SIN = (
    0.999882465186241, -0.1662326327675864, 0.00808644586820865,
    -0.00015325191256653362,
)


def _sincos(ph):
    y = ph * ph
    c = jnp.float32(_COS[-1])
    for k in reversed(_COS[:-1]):
        c = c * y + jnp.float32(k)
    s = jnp.float32(_SIN[-1])
    for k in reversed(_SIN[:-1]):
        s = s * y + jnp.float32(k)
    return s * ph, c


def _body(h_hbm, r_hbm, t_hbm, er_hbm, ei_hbm, rel_hbm, gam_hbm, out_hbm,
          hidx, ridx, tidx,
          hre0, hre1, him0, him1, rph0, rph1, tre0, tre1, tim0, tim1,
          gam, outv, stag, sem0, sem1):
    cid = lax.axis_index("c")
    sid = lax.axis_index("s")
    wid = sid * NC + cid
    base = wid * SPW
    row_refs = ((hre0, him0, rph0, tre0, tim0),
                (hre1, him1, rph1, tre1, tim1))
    sems = (sem0, sem1)

    pltpu.sync_copy(gam_hbm, gam)
    pltpu.sync_copy(h_hbm.at[pl.ds(base, SPW)], hidx)
    pltpu.sync_copy(r_hbm.at[pl.ds(base, SPW)], ridx)
    pltpu.sync_copy(t_hbm.at[pl.ds(base, SPW)], tidx)
    neg_gamma = -gam[...]

    def fetch(k, slot):
        sl = pl.ds(k * C, C)
        hre, him, rph, tre, tim = row_refs[slot]
        sem = sems[slot]
        cps = []
        H = C // 2
        for p in range(2):
            slp = pl.ds(k * C + p * H, H)
            dst = pl.ds(p * H, H)
            cps += [
                pltpu.async_copy(er_hbm.at[hidx.at[slp]], hre.at[dst], sem),
                pltpu.async_copy(ei_hbm.at[hidx.at[slp]], him.at[dst], sem),
                pltpu.async_copy(rel_hbm.at[ridx.at[slp]], rph.at[dst], sem),
                pltpu.async_copy(er_hbm.at[tidx.at[slp]], tre.at[dst], sem),
                pltpu.async_copy(ei_hbm.at[tidx.at[slp]], tim.at[dst], sem),
            ]
        return cps

    iota = lax.iota(jnp.int32, 16)

    def compute(k, slot):
        hre, him, rph, tre, tim = row_refs[slot]

        def sbody(i, _):
            acc = jnp.zeros((16,), jnp.float32)
            for dg in range(DIM // 16):
                sl = pl.ds(dg * 16, 16)
                ph = rph[i, sl]
                a = hre[i, sl]
                b = him[i, sl]
                cre = tre[i, sl]
                cim = tim[i, sl]
                sv, cv = _sincos(ph)
                re = a * cv - b * sv - cre
                im = a * sv + b * cv - cim
                acc = acc + (jnp.abs(re) + jnp.abs(im))
            stag[i, :] = acc
            return jnp.int32(0)

        lax.fori_loop(0, C, sbody, jnp.int32(0), unroll=2)

        for g in range(GROUPS):
            rows = iota + jnp.int32(g * 16)
            tot = neg_gamma
            for r in range(16):
                cols = (iota + jnp.int32(r)) & jnp.int32(15)
                v = plsc.load_gather(stag, [rows, cols])
                tot = tot + v
            outv[pl.ds(k * C + g * 16, 16)] = tot

    cps = fetch(0, 0)
    for k in range(NCHUNK):
        nxt = fetch(k + 1, (k + 1) % 2) if k + 1 < NCHUNK else None
        for cp in cps:
            cp.wait()
        compute(k, k % 2)
        cps = nxt

    pltpu.sync_copy(outv, out_hbm.at[pl.ds(base, SPW)])


@jax.jit
def _rotate_scores(h, r, t, er, ei, rel, gam_vec):
    mesh = plsc.VectorSubcoreMesh(
        core_axis_name="c", subcore_axis_name="s", num_cores=NC,
        num_subcores=NS)
    grid = functools.partial(
        pl.kernel,
        out_type=jax.ShapeDtypeStruct((B,), jnp.float32),
        mesh=mesh,
        compiler_params=pltpu.CompilerParams(needs_layout_passes=False),
        scratch_types=(
            [pltpu.VMEM((SPW,), jnp.int32)] * 3
            + [pltpu.VMEM((C, DIM), jnp.float32)] * 10
            + [pltpu.VMEM((16,), jnp.float32),
               pltpu.VMEM((SPW,), jnp.float32),
               pltpu.VMEM((C, 16), jnp.float32),
               pltpu.SemaphoreType.DMA,
               pltpu.SemaphoreType.DMA]
        ),
    )
    return grid(_body)(h, r, t, er, ei, rel, gam_vec)


def kernel(pos_sample, ent_embd, ent_embd_im, rel_embd, gamma):
    h = pos_sample[:, 0].astype(jnp.int32)
    r = pos_sample[:, 1].astype(jnp.int32)
    t = pos_sample[:, 2].astype(jnp.int32)
    gam_vec = jnp.full((16,), gamma, jnp.float32)
    scores = _rotate_scores(h, r, t, ent_embd, ent_embd_im, rel_embd, gam_vec)
    return scores.reshape(B, 1)


# speedup vs baseline: 1.1160x; 1.0117x over previous
import functools

import jax
import jax.numpy as jnp
from jax import lax
from jax.experimental import pallas as pl
from jax.experimental.pallas import tpu as pltpu
from jax.experimental.pallas import tpu_sc as plsc

B = 16384
DIM = 128
NC = 2
NS = 16
NW = NC * NS
SPW = B // NW
C = 64
NCHUNK = SPW // C
GROUPS = C // 16

_COS = (
    0.9989871519760842, -0.49624862730581776, 0.0395223027568335,
    -0.0009928615940640857,
)
_SIN = (
    0.999882465186241, -0.1662326327675864, 0.00808644586820865,
    -0.00015325191256653362,
)


def _sincos(ph):
    y = ph * ph
    c = jnp.float32(_COS[-1])
    for k in reversed(_COS[:-1]):
        c = c * y + jnp.float32(k)
    s = jnp.float32(_SIN[-1])
    for k in reversed(_SIN[:-1]):
        s = s * y + jnp.float32(k)
    return s * ph, c


def _body(h_hbm, r_hbm, t_hbm, er_hbm, ei_hbm, rel_hbm, gam_hbm, out_hbm,
          hidx, ridx, tidx,
          hre0, hre1, him0, him1, rph0, rph1, tre0, tre1, tim0, tim1,
          gam, outv, stag, sem0, sem1):
    cid = lax.axis_index("c")
    sid = lax.axis_index("s")
    wid = sid * NC + cid
    base = wid * SPW
    row_refs = ((hre0, him0, rph0, tre0, tim0),
                (hre1, him1, rph1, tre1, tim1))
    sems = (sem0, sem1)

    pltpu.sync_copy(gam_hbm, gam)
    pltpu.sync_copy(h_hbm.at[pl.ds(base, SPW)], hidx)
    pltpu.sync_copy(r_hbm.at[pl.ds(base, SPW)], ridx)
    pltpu.sync_copy(t_hbm.at[pl.ds(base, SPW)], tidx)
    neg_gamma = -gam[...]

    def fetch(k, slot):
        sl = pl.ds(k * C, C)
        hre, him, rph, tre, tim = row_refs[slot]
        sem = sems[slot]
        return [
            pltpu.async_copy(er_hbm.at[hidx.at[sl]], hre, sem),
            pltpu.async_copy(ei_hbm.at[hidx.at[sl]], him, sem),
            pltpu.async_copy(rel_hbm.at[ridx.at[sl]], rph, sem),
            pltpu.async_copy(er_hbm.at[tidx.at[sl]], tre, sem),
            pltpu.async_copy(ei_hbm.at[tidx.at[sl]], tim, sem),
        ]

    iota = lax.iota(jnp.int32, 16)

    def compute(k, slot):
        hre, him, rph, tre, tim = row_refs[slot]

        def sbody(i, _):
            acc = jnp.zeros((16,), jnp.float32)
            for dg in range(DIM // 16):
                sl = pl.ds(dg * 16, 16)
                ph = rph[i, sl]
                a = hre[i, sl]
                b = him[i, sl]
                cre = tre[i, sl]
                cim = tim[i, sl]
                sv, cv = _sincos(ph)
                re = a * cv - b * sv - cre
                im = a * sv + b * cv - cim
                acc = acc + (jnp.abs(re) + jnp.abs(im))
            stag[i, :] = acc
            return jnp.int32(0)

        lax.fori_loop(0, C, sbody, jnp.int32(0), unroll=2)

        for g in range(GROUPS):
            rows = iota + jnp.int32(g * 16)
            tot = neg_gamma
            for r in range(16):
                cols = (iota + jnp.int32(r)) & jnp.int32(15)
                v = plsc.load_gather(stag, [rows, cols])
                tot = tot + v
            outv[pl.ds(k * C + g * 16, 16)] = tot

    cps = fetch(0, 0)
    for k in range(NCHUNK):
        nxt = fetch(k + 1, (k + 1) % 2) if k + 1 < NCHUNK else None
        for cp in cps:
            cp.wait()
        compute(k, k % 2)
        cps = nxt

    pltpu.sync_copy(outv, out_hbm.at[pl.ds(base, SPW)])


@jax.jit
def _rotate_scores(h, r, t, er, ei, rel, gam_vec):
    mesh = plsc.VectorSubcoreMesh(
        core_axis_name="c", subcore_axis_name="s", num_cores=NC,
        num_subcores=NS)
    grid = functools.partial(
        pl.kernel,
        out_type=jax.ShapeDtypeStruct((B,), jnp.float32),
        mesh=mesh,
        compiler_params=pltpu.CompilerParams(needs_layout_passes=False),
        scratch_types=(
            [pltpu.VMEM((SPW,), jnp.int32)] * 3
            + [pltpu.VMEM((C, DIM), jnp.float32)] * 10
            + [pltpu.VMEM((16,), jnp.float32),
               pltpu.VMEM((SPW,), jnp.float32),
               pltpu.VMEM((C, 16), jnp.float32),
               pltpu.SemaphoreType.DMA,
               pltpu.SemaphoreType.DMA]
        ),
    )
    return grid(_body)(h, r, t, er, ei, rel, gam_vec)


def kernel(pos_sample, ent_embd, ent_embd_im, rel_embd, gamma):
    h = pos_sample[:, 0].astype(jnp.int32)
    r = pos_sample[:, 1].astype(jnp.int32)
    t = pos_sample[:, 2].astype(jnp.int32)
    gam_vec = jnp.full((16,), gamma, jnp.float32)
    scores = _rotate_scores(h, r, t, ent_embd, ent_embd_im, rel_embd, gam_vec)
    return scores.reshape(B, 1)


# speedup vs baseline: 1.1401x; 1.0216x over previous
import functools

import jax
import jax.numpy as jnp
from jax import lax
from jax.experimental import pallas as pl
from jax.experimental.pallas import tpu as pltpu
from jax.experimental.pallas import tpu_sc as plsc

B = 16384
DIM = 128
NC = 2
NS = 16
NW = NC * NS
SPW = B // NW
C = 64
NCHUNK = SPW // C
GROUPS = C // 16

_COS = (
    0.9989871519760842, -0.49624862730581776, 0.0395223027568335,
    -0.0009928615940640857,
)
_SIN = (
    0.999882465186241, -0.1662326327675864, 0.00808644586820865,
    -0.00015325191256653362,
)


def _sincos(ph):
    y = ph * ph
    c = jnp.float32(_COS[-1])
    for k in reversed(_COS[:-1]):
        c = c * y + jnp.float32(k)
    s = jnp.float32(_SIN[-1])
    for k in reversed(_SIN[:-1]):
        s = s * y + jnp.float32(k)
    return s * ph, c


def _body(h_hbm, r_hbm, t_hbm, er_hbm, ei_hbm, rel_hbm, gam_hbm, out_hbm,
          hidx, ridx, tidx,
          hre0, hre1, him0, him1, rph0, rph1, tre0, tre1, tim0, tim1,
          gam, outv, stag, sem0, sem1):
    cid = lax.axis_index("c")
    sid = lax.axis_index("s")
    wid = sid * NC + cid
    base = wid * SPW
    row_refs = ((hre0, him0, rph0, tre0, tim0),
                (hre1, him1, rph1, tre1, tim1))
    sems = (sem0, sem1)

    pre = [
        pltpu.async_copy(gam_hbm, gam, sem0),
        pltpu.async_copy(h_hbm.at[pl.ds(base, SPW)], hidx, sem0),
        pltpu.async_copy(r_hbm.at[pl.ds(base, SPW)], ridx, sem0),
        pltpu.async_copy(t_hbm.at[pl.ds(base, SPW)], tidx, sem0),
    ]
    for cp in pre:
        cp.wait()
    neg_gamma = -gam[...]

    def fetch(k, slot):
        sl = pl.ds(k * C, C)
        hre, him, rph, tre, tim = row_refs[slot]
        sem = sems[slot]
        return [
            pltpu.async_copy(er_hbm.at[hidx.at[sl]], hre, sem),
            pltpu.async_copy(ei_hbm.at[hidx.at[sl]], him, sem),
            pltpu.async_copy(rel_hbm.at[ridx.at[sl]], rph, sem),
            pltpu.async_copy(er_hbm.at[tidx.at[sl]], tre, sem),
            pltpu.async_copy(ei_hbm.at[tidx.at[sl]], tim, sem),
        ]

    iota = lax.iota(jnp.int32, 16)

    def compute(k, slot):
        hre, him, rph, tre, tim = row_refs[slot]

        def sbody(i, _):
            acc = jnp.zeros((16,), jnp.float32)
            for dg in range(DIM // 16):
                sl = pl.ds(dg * 16, 16)
                ph = rph[i, sl]
                a = hre[i, sl]
                b = him[i, sl]
                cre = tre[i, sl]
                cim = tim[i, sl]
                sv, cv = _sincos(ph)
                re = a * cv - b * sv - cre
                im = a * sv + b * cv - cim
                acc = acc + (jnp.abs(re) + jnp.abs(im))
            stag[i, :] = acc
            return jnp.int32(0)

        lax.fori_loop(0, C, sbody, jnp.int32(0), unroll=2)

        for g in range(GROUPS):
            rows = iota + jnp.int32(g * 16)
            tot = neg_gamma
            for r in range(16):
                cols = (iota + jnp.int32(r)) & jnp.int32(15)
                v = plsc.load_gather(stag, [rows, cols])
                tot = tot + v
            outv[pl.ds(k * C + g * 16, 16)] = tot

    cps = fetch(0, 0)
    for k in range(NCHUNK):
        nxt = fetch(k + 1, (k + 1) % 2) if k + 1 < NCHUNK else None
        for cp in cps:
            cp.wait()
        compute(k, k % 2)
        cps = nxt

    pltpu.sync_copy(outv, out_hbm.at[pl.ds(base, SPW)])


@jax.jit
def _rotate_scores(h, r, t, er, ei, rel, gam_vec):
    mesh = plsc.VectorSubcoreMesh(
        core_axis_name="c", subcore_axis_name="s", num_cores=NC,
        num_subcores=NS)
    grid = functools.partial(
        pl.kernel,
        out_type=jax.ShapeDtypeStruct((B,), jnp.float32),
        mesh=mesh,
        compiler_params=pltpu.CompilerParams(needs_layout_passes=False),
        scratch_types=(
            [pltpu.VMEM((SPW,), jnp.int32)] * 3
            + [pltpu.VMEM((C, DIM), jnp.float32)] * 10
            + [pltpu.VMEM((16,), jnp.float32),
               pltpu.VMEM((SPW,), jnp.float32),
               pltpu.VMEM((C, 16), jnp.float32),
               pltpu.SemaphoreType.DMA,
               pltpu.SemaphoreType.DMA]
        ),
    )
    return grid(_body)(h, r, t, er, ei, rel, gam_vec)


def kernel(pos_sample, ent_embd, ent_embd_im, rel_embd, gamma):
    h = pos_sample[:, 0].astype(jnp.int32)
    r = pos_sample[:, 1].astype(jnp.int32)
    t = pos_sample[:, 2].astype(jnp.int32)
    gam_vec = jnp.full((16,), gamma, jnp.float32)
    scores = _rotate_scores(h, r, t, ent_embd, ent_embd_im, rel_embd, gam_vec)
    return scores.reshape(B, 1)


# speedup vs baseline: 1.1451x; 1.0044x over previous
import functools

import jax
import jax.numpy as jnp
from jax import lax
from jax.experimental import pallas as pl
from jax.experimental.pallas import tpu as pltpu
from jax.experimental.pallas import tpu_sc as plsc

B = 16384
DIM = 128
NC = 2
NS = 16
NW = NC * NS
SPW = B // NW
C = 64
NCHUNK = SPW // C
GROUPS = C // 16

_COS = (
    0.9989871519760842, -0.49624862730581776, 0.0395223027568335,
    -0.0009928615940640857,
)
_SIN = (
    0.999882465186241, -0.1662326327675864, 0.00808644586820865,
    -0.00015325191256653362,
)


def _sincos(ph):
    y = ph * ph
    c = jnp.float32(_COS[-1])
    for k in reversed(_COS[:-1]):
        c = c * y + jnp.float32(k)
    s = jnp.float32(_SIN[-1])
    for k in reversed(_SIN[:-1]):
        s = s * y + jnp.float32(k)
    return s * ph, c


def _body(h_hbm, r_hbm, t_hbm, er_hbm, ei_hbm, rel_hbm, gam_hbm, out_hbm,
          hidx, ridx, tidx,
          hre0, hre1, him0, him1, rph0, rph1, tre0, tre1, tim0, tim1,
          gam, outv, stag, sem0, sem1, osem0, osem1):
    cid = lax.axis_index("c")
    sid = lax.axis_index("s")
    wid = sid * NC + cid
    base = wid * SPW
    row_refs = ((hre0, him0, rph0, tre0, tim0),
                (hre1, him1, rph1, tre1, tim1))
    sems = (sem0, sem1)

    pre = [
        pltpu.async_copy(gam_hbm, gam, sem0),
        pltpu.async_copy(h_hbm.at[pl.ds(base, SPW)], hidx, sem0),
        pltpu.async_copy(r_hbm.at[pl.ds(base, SPW)], ridx, sem0),
        pltpu.async_copy(t_hbm.at[pl.ds(base, SPW)], tidx, sem0),
    ]
    for cp in pre:
        cp.wait()
    neg_gamma = -gam[...]

    def fetch(k, slot):
        sl = pl.ds(k * C, C)
        hre, him, rph, tre, tim = row_refs[slot]
        sem = sems[slot]
        return [
            pltpu.async_copy(er_hbm.at[hidx.at[sl]], hre, sem),
            pltpu.async_copy(ei_hbm.at[hidx.at[sl]], him, sem),
            pltpu.async_copy(rel_hbm.at[ridx.at[sl]], rph, sem),
            pltpu.async_copy(er_hbm.at[tidx.at[sl]], tre, sem),
            pltpu.async_copy(ei_hbm.at[tidx.at[sl]], tim, sem),
        ]

    iota = lax.iota(jnp.int32, 16)

    def compute(k, slot):
        hre, him, rph, tre, tim = row_refs[slot]

        def sbody(i, _):
            acc = jnp.zeros((16,), jnp.float32)
            for dg in range(DIM // 16):
                sl = pl.ds(dg * 16, 16)
                ph = rph[i, sl]
                a = hre[i, sl]
                b = him[i, sl]
                cre = tre[i, sl]
                cim = tim[i, sl]
                sv, cv = _sincos(ph)
                re = a * cv - b * sv - cre
                im = a * sv + b * cv - cim
                acc = acc + (jnp.abs(re) + jnp.abs(im))
            stag[i, :] = acc
            return jnp.int32(0)

        lax.fori_loop(0, C, sbody, jnp.int32(0), unroll=2)

        for g in range(GROUPS):
            rows = iota + jnp.int32(g * 16)
            tot = neg_gamma
            for r in range(16):
                cols = (iota + jnp.int32(r)) & jnp.int32(15)
                v = plsc.load_gather(stag, [rows, cols])
                tot = tot + v
            outv[slot, pl.ds(g * 16, 16)] = tot
        return pltpu.async_copy(
            outv.at[slot], out_hbm.at[pl.ds(base + k * C, C)],
            osem0 if slot == 0 else osem1)

    cps = fetch(0, 0)
    ocps = [None, None]
    for k in range(NCHUNK):
        nxt = fetch(k + 1, (k + 1) % 2) if k + 1 < NCHUNK else None
        for cp in cps:
            cp.wait()
        if ocps[k % 2] is not None:
            ocps[k % 2].wait()
        ocps[k % 2] = compute(k, k % 2)
        cps = nxt
    for ocp in ocps:
        if ocp is not None:
            ocp.wait()


@jax.jit
def _rotate_scores(h, r, t, er, ei, rel, gam_vec):
    mesh = plsc.VectorSubcoreMesh(
        core_axis_name="c", subcore_axis_name="s", num_cores=NC,
        num_subcores=NS)
    grid = functools.partial(
        pl.kernel,
        out_type=jax.ShapeDtypeStruct((B,), jnp.float32),
        mesh=mesh,
        compiler_params=pltpu.CompilerParams(needs_layout_passes=False),
        scratch_types=(
            [pltpu.VMEM((SPW,), jnp.int32)] * 3
            + [pltpu.VMEM((C, DIM), jnp.float32)] * 10
            + [pltpu.VMEM((16,), jnp.float32),
               pltpu.VMEM((2, C), jnp.float32),
               pltpu.VMEM((C, 16), jnp.float32),
               pltpu.SemaphoreType.DMA,
               pltpu.SemaphoreType.DMA,
               pltpu.SemaphoreType.DMA,
               pltpu.SemaphoreType.DMA]
        ),
    )
    return grid(_body)(h, r, t, er, ei, rel, gam_vec)


def kernel(pos_sample, ent_embd, ent_embd_im, rel_embd, gamma):
    h = pos_sample[:, 0].astype(jnp.int32)
    r = pos_sample[:, 1].astype(jnp.int32)
    t = pos_sample[:, 2].astype(jnp.int32)
    gam_vec = jnp.full((16,), gamma, jnp.float32)
    scores = _rotate_scores(h, r, t, ent_embd, ent_embd_im, rel_embd, gam_vec)
    return scores.reshape(B, 1)
